# Initial kernel scaffold; baseline (speedup 1.0000x reference)
import jax
import jax.numpy as jnp
from jax.experimental import pallas as pl


def kernel(residual, down, buffers, W_proj, b_proj, W_skip, b_skip):
    raise NotImplementedError("write your pallas kernel here")


# speedup vs baseline: 1.3731x; 1.3731x over previous
import functools

import jax
import jax.numpy as jnp
from jax import lax
from jax.experimental import pallas as pl
from jax.experimental.pallas import tpu as pltpu
from jax.experimental.pallas import tpu_sc as plsc

N_FINE = 100000
N_COARSE = 50000
IN_CH = 256
SKIP_CH = 128
OUT_CH = 256

NUM_CORES = 2
NUM_SUBCORES = 16
NUM_WORKERS = NUM_CORES * NUM_SUBCORES

CHUNK = 80
N_CHUNKS = N_FINE // CHUNK
CHUNKS_PER_WORKER = -(-N_CHUNKS // NUM_WORKERS)


def _sc_gather_body(idx_hbm, down_hbm, out_hbm, idx_v, rows_v, sem):
    wid = lax.axis_index("s") * NUM_CORES + lax.axis_index("c")

    def step(i, carry):
        c = wid + i * NUM_WORKERS

        @pl.when(c < N_CHUNKS)
        def _():
            pltpu.sync_copy(idx_hbm.at[c], idx_v)
            pltpu.async_copy(down_hbm.at[idx_v], rows_v, sem).wait()
            pltpu.sync_copy(rows_v, out_hbm.at[pl.ds(c * CHUNK, CHUNK)])

        return carry

    lax.fori_loop(0, CHUNKS_PER_WORKER, step, 0)


_sc_gather = pl.kernel(
    _sc_gather_body,
    out_type=jax.ShapeDtypeStruct((N_FINE, IN_CH), jnp.float32),
    mesh=plsc.VectorSubcoreMesh(core_axis_name="c", subcore_axis_name="s"),
    scratch_types=[
        pltpu.VMEM((CHUNK,), jnp.int32),
        pltpu.VMEM((CHUNK, IN_CH), jnp.float32),
        pltpu.SemaphoreType.DMA,
    ],
)


def _tc_fused_body(res_ref, gat_ref, wskip_ref, wproj_ref, bias_ref, out_ref):
    out_ref[...] = (
        jnp.dot(res_ref[...], wskip_ref[...], preferred_element_type=jnp.float32)
        + jnp.dot(gat_ref[...], wproj_ref[...], preferred_element_type=jnp.float32)
        + bias_ref[...]
    )


ROWS_BLK = 1000
GRID = N_FINE // ROWS_BLK

_tc_fused = pl.pallas_call(
    _tc_fused_body,
    grid=(GRID,),
    in_specs=[
        pl.BlockSpec((ROWS_BLK, SKIP_CH), lambda i: (i, 0)),
        pl.BlockSpec((ROWS_BLK, IN_CH), lambda i: (i, 0)),
        pl.BlockSpec((SKIP_CH, OUT_CH), lambda i: (0, 0)),
        pl.BlockSpec((IN_CH, OUT_CH), lambda i: (0, 0)),
        pl.BlockSpec((1, OUT_CH), lambda i: (0, 0)),
    ],
    out_specs=pl.BlockSpec((ROWS_BLK, OUT_CH), lambda i: (i, 0)),
    out_shape=jax.ShapeDtypeStruct((N_FINE, OUT_CH), jnp.float32),
)


def kernel(residual, down, buffers, W_proj, b_proj, W_skip, b_skip):
    idx = buffers.reshape(N_CHUNKS, CHUNK)
    gathered = _sc_gather(idx, down)
    bias = (b_proj + b_skip).reshape(1, OUT_CH)
    return _tc_fused(residual, gathered, W_skip, W_proj, bias)


# speedup vs baseline: 1.5903x; 1.1582x over previous
import functools

import jax
import jax.numpy as jnp
from jax import lax
from jax.experimental import pallas as pl
from jax.experimental.pallas import tpu as pltpu
from jax.experimental.pallas import tpu_sc as plsc

N_FINE = 100000
N_COARSE = 50000
IN_CH = 256
SKIP_CH = 128
OUT_CH = 256

NUM_CORES = 2
NUM_SUBCORES = 16
NUM_WORKERS = NUM_CORES * NUM_SUBCORES

CHUNK = 80
N_CHUNKS = N_FINE // CHUNK
SLOTS = 40


def _sc_gather_body(idx_hbm, down_hbm, out_hbm, idx_all, rows0, rows1,
                    sem_g0, sem_g1, sem_w0, sem_w1):
    wid = lax.axis_index("s") * NUM_CORES + lax.axis_index("c")

    def gather(i, rows, sem):
        return pltpu.make_async_copy(down_hbm.at[idx_all.at[i]], rows, sem)

    def writeback(i, rows, sem):
        c = wid + i * NUM_WORKERS
        return pltpu.make_async_copy(rows, out_hbm.at[pl.ds(c * CHUNK, CHUNK)], sem)

    def valid(i):
        return wid + i * NUM_WORKERS < N_CHUNKS

    pltpu.sync_copy(idx_hbm.at[:, wid], idx_all)
    gather(0, rows0, sem_g0).start()

    def step(t, carry):
        i = 2 * t
        gather(i, rows0, sem_g0).wait()
        writeback(i, rows0, sem_w0).start()

        @pl.when(valid(i + 1))
        def _():
            @pl.when(t > 0)
            def _():
                writeback(i - 1, rows1, sem_w1).wait()

            gather(i + 1, rows1, sem_g1).start()

        @pl.when(valid(i + 1))
        def _():
            gather(i + 1, rows1, sem_g1).wait()
            writeback(i + 1, rows1, sem_w1).start()

        @pl.when(valid(i + 2))
        def _():
            writeback(i, rows0, sem_w0).wait()
            gather(i + 2, rows0, sem_g0).start()

        return carry

    lax.fori_loop(0, SLOTS // 2, step, 0)

    writeback(0, rows0, sem_w0).wait()
    writeback(0, rows1, sem_w1).wait()


_sc_gather = pl.kernel(
    _sc_gather_body,
    out_type=jax.ShapeDtypeStruct((N_FINE, IN_CH), jnp.float32),
    mesh=plsc.VectorSubcoreMesh(core_axis_name="c", subcore_axis_name="s"),
    scratch_types=[
        pltpu.VMEM((SLOTS, CHUNK), jnp.int32),
        pltpu.VMEM((CHUNK, IN_CH), jnp.float32),
        pltpu.VMEM((CHUNK, IN_CH), jnp.float32),
        pltpu.SemaphoreType.DMA,
        pltpu.SemaphoreType.DMA,
        pltpu.SemaphoreType.DMA,
        pltpu.SemaphoreType.DMA,
    ],
)


def _tc_fused_body(res_ref, gat_ref, wskip_ref, wproj_ref, bias_ref, out_ref):
    out_ref[...] = (
        jnp.dot(res_ref[...], wskip_ref[...], preferred_element_type=jnp.float32)
        + jnp.dot(gat_ref[...], wproj_ref[...], preferred_element_type=jnp.float32)
        + bias_ref[...]
    )


ROWS_BLK = 1000
GRID = N_FINE // ROWS_BLK

_tc_fused = pl.pallas_call(
    _tc_fused_body,
    grid=(GRID,),
    in_specs=[
        pl.BlockSpec((ROWS_BLK, SKIP_CH), lambda i: (i, 0)),
        pl.BlockSpec((ROWS_BLK, IN_CH), lambda i: (i, 0)),
        pl.BlockSpec((SKIP_CH, OUT_CH), lambda i: (0, 0)),
        pl.BlockSpec((IN_CH, OUT_CH), lambda i: (0, 0)),
        pl.BlockSpec((1, OUT_CH), lambda i: (0, 0)),
    ],
    out_specs=pl.BlockSpec((ROWS_BLK, OUT_CH), lambda i: (i, 0)),
    out_shape=jax.ShapeDtypeStruct((N_FINE, OUT_CH), jnp.float32),
)


def kernel(residual, down, buffers, W_proj, b_proj, W_skip, b_skip):
    idx = jnp.pad(buffers, (0, SLOTS * NUM_WORKERS * CHUNK - N_FINE))
    idx = idx.reshape(SLOTS, NUM_WORKERS, CHUNK)
    gathered = _sc_gather(idx, down)
    bias = (b_proj + b_skip).reshape(1, OUT_CH)
    return _tc_fused(residual, gathered, W_skip, W_proj, bias)


# speedup vs baseline: 1.5922x; 1.0012x over previous
import functools

import jax
import jax.numpy as jnp
from jax import lax
from jax.experimental import pallas as pl
from jax.experimental.pallas import tpu as pltpu
from jax.experimental.pallas import tpu_sc as plsc

N_FINE = 100000
N_COARSE = 50000
IN_CH = 256
SKIP_CH = 128
OUT_CH = 256

NUM_CORES = 2
NUM_SUBCORES = 16
NUM_WORKERS = NUM_CORES * NUM_SUBCORES

CHUNK = 80
N_CHUNKS = N_FINE // CHUNK
SLOTS = 40


def _sc_gather_body(idx_hbm, down_hbm, out_hbm, idx_all, rows0, rows1,
                    sem_g0, sem_g1, sem_w0, sem_w1):
    wid = lax.axis_index("s") * NUM_CORES + lax.axis_index("c")

    def gather(i, rows, sem):
        return pltpu.make_async_copy(down_hbm.at[idx_all.at[i]], rows, sem)

    def writeback(i, rows, sem):
        c = wid + i * NUM_WORKERS
        return pltpu.make_async_copy(rows, out_hbm.at[pl.ds(c * CHUNK, CHUNK)], sem)

    def valid(i):
        return wid + i * NUM_WORKERS < N_CHUNKS

    pltpu.sync_copy(idx_hbm.at[:, wid], idx_all)
    gather(0, rows0, sem_g0).start()

    def step(t, carry):
        i = 2 * t
        gather(i, rows0, sem_g0).wait()
        writeback(i, rows0, sem_w0).start()

        @pl.when(valid(i + 1))
        def _():
            @pl.when(t > 0)
            def _():
                writeback(i - 1, rows1, sem_w1).wait()

            gather(i + 1, rows1, sem_g1).start()

        @pl.when(valid(i + 1))
        def _():
            gather(i + 1, rows1, sem_g1).wait()
            writeback(i + 1, rows1, sem_w1).start()

        @pl.when(valid(i + 2))
        def _():
            writeback(i, rows0, sem_w0).wait()
            gather(i + 2, rows0, sem_g0).start()

        return carry

    lax.fori_loop(0, SLOTS // 2, step, 0)

    writeback(0, rows0, sem_w0).wait()
    writeback(0, rows1, sem_w1).wait()


_sc_gather = pl.kernel(
    _sc_gather_body,
    out_type=jax.ShapeDtypeStruct((N_FINE, IN_CH), jnp.float32),
    mesh=plsc.VectorSubcoreMesh(core_axis_name="c", subcore_axis_name="s"),
    scratch_types=[
        pltpu.VMEM((SLOTS, CHUNK), jnp.int32),
        pltpu.VMEM((CHUNK, IN_CH), jnp.float32),
        pltpu.VMEM((CHUNK, IN_CH), jnp.float32),
        pltpu.SemaphoreType.DMA,
        pltpu.SemaphoreType.DMA,
        pltpu.SemaphoreType.DMA,
        pltpu.SemaphoreType.DMA,
    ],
)


def _tc_fused_body(res_ref, gat_ref, wskip_ref, wproj_ref, bias_ref, out_ref):
    out_ref[...] = (
        jnp.dot(
            res_ref[...].astype(jnp.bfloat16),
            wskip_ref[...].astype(jnp.bfloat16),
            preferred_element_type=jnp.float32,
        )
        + jnp.dot(
            gat_ref[...].astype(jnp.bfloat16),
            wproj_ref[...].astype(jnp.bfloat16),
            preferred_element_type=jnp.float32,
        )
        + bias_ref[...]
    )


ROWS_BLK = 1000
GRID = N_FINE // ROWS_BLK

_tc_fused = pl.pallas_call(
    _tc_fused_body,
    grid=(GRID,),
    in_specs=[
        pl.BlockSpec((ROWS_BLK, SKIP_CH), lambda i: (i, 0)),
        pl.BlockSpec((ROWS_BLK, IN_CH), lambda i: (i, 0)),
        pl.BlockSpec((SKIP_CH, OUT_CH), lambda i: (0, 0)),
        pl.BlockSpec((IN_CH, OUT_CH), lambda i: (0, 0)),
        pl.BlockSpec((1, OUT_CH), lambda i: (0, 0)),
    ],
    out_specs=pl.BlockSpec((ROWS_BLK, OUT_CH), lambda i: (i, 0)),
    out_shape=jax.ShapeDtypeStruct((N_FINE, OUT_CH), jnp.float32),
)


def kernel(residual, down, buffers, W_proj, b_proj, W_skip, b_skip):
    idx = jnp.pad(buffers, (0, SLOTS * NUM_WORKERS * CHUNK - N_FINE))
    idx = idx.reshape(SLOTS, NUM_WORKERS, CHUNK)
    gathered = _sc_gather(idx, down)
    bias = (b_proj + b_skip).reshape(1, OUT_CH)
    return _tc_fused(residual, gathered, W_skip, W_proj, bias)


# speedup vs baseline: 1.8602x; 1.1684x over previous
import functools

import jax
import jax.numpy as jnp
from jax import lax
from jax.experimental import pallas as pl
from jax.experimental.pallas import tpu as pltpu
from jax.experimental.pallas import tpu_sc as plsc

N_FINE = 100000
N_COARSE = 50000
IN_CH = 256
SKIP_CH = 128
OUT_CH = 256

NUM_CORES = 2
NUM_SUBCORES = 16
NUM_WORKERS = NUM_CORES * NUM_SUBCORES

CHUNK = 80
N_CHUNKS = N_FINE // CHUNK
SLOTS = 40


def _sc_gather_body(idx_hbm, down_hbm, out_hbm, idx_all, rows0, rows1,
                    sem_g0, sem_g1, sem_w0, sem_w1):
    wid = lax.axis_index("s") * NUM_CORES + lax.axis_index("c")

    def gather(i, rows, sem):
        return pltpu.make_async_copy(down_hbm.at[idx_all.at[i]], rows, sem)

    def writeback(i, rows, sem):
        c = wid + i * NUM_WORKERS
        return pltpu.make_async_copy(rows, out_hbm.at[pl.ds(c * CHUNK, CHUNK)], sem)

    def valid(i):
        return wid + i * NUM_WORKERS < N_CHUNKS

    pltpu.sync_copy(idx_hbm.at[:, wid], idx_all)
    gather(0, rows0, sem_g0).start()

    def step(t, carry):
        i = 2 * t
        gather(i, rows0, sem_g0).wait()
        writeback(i, rows0, sem_w0).start()

        @pl.when(valid(i + 1))
        def _():
            @pl.when(t > 0)
            def _():
                writeback(i - 1, rows1, sem_w1).wait()

            gather(i + 1, rows1, sem_g1).start()

        @pl.when(valid(i + 1))
        def _():
            gather(i + 1, rows1, sem_g1).wait()
            writeback(i + 1, rows1, sem_w1).start()

        @pl.when(valid(i + 2))
        def _():
            writeback(i, rows0, sem_w0).wait()
            gather(i + 2, rows0, sem_g0).start()

        return carry

    lax.fori_loop(0, SLOTS // 2, step, 0)

    writeback(0, rows0, sem_w0).wait()
    writeback(0, rows1, sem_w1).wait()


_sc_gather = pl.kernel(
    _sc_gather_body,
    out_type=jax.ShapeDtypeStruct((N_FINE, IN_CH), jnp.float32),
    mesh=plsc.VectorSubcoreMesh(core_axis_name="c", subcore_axis_name="s"),
    scratch_types=[
        pltpu.VMEM((SLOTS, CHUNK), jnp.int32),
        pltpu.VMEM((CHUNK, IN_CH), jnp.float32),
        pltpu.VMEM((CHUNK, IN_CH), jnp.float32),
        pltpu.SemaphoreType.DMA,
        pltpu.SemaphoreType.DMA,
        pltpu.SemaphoreType.DMA,
        pltpu.SemaphoreType.DMA,
    ],
)


def _tc_fused_body(res_ref, gat_ref, wskip_ref, wproj_ref, bias_ref, out_ref):
    out_ref[...] = (
        jnp.dot(res_ref[...], wskip_ref[...], preferred_element_type=jnp.float32)
        + jnp.dot(gat_ref[...], wproj_ref[...], preferred_element_type=jnp.float32)
        + bias_ref[...]
    )


ROWS_BLK = 2000
GRID = N_FINE // ROWS_BLK

_tc_fused = pl.pallas_call(
    _tc_fused_body,
    grid=(GRID,),
    in_specs=[
        pl.BlockSpec((ROWS_BLK, SKIP_CH), lambda i: (i, 0)),
        pl.BlockSpec((ROWS_BLK, IN_CH), lambda i: (i, 0)),
        pl.BlockSpec((SKIP_CH, OUT_CH), lambda i: (0, 0)),
        pl.BlockSpec((IN_CH, OUT_CH), lambda i: (0, 0)),
        pl.BlockSpec((1, OUT_CH), lambda i: (0, 0)),
    ],
    out_specs=pl.BlockSpec((ROWS_BLK, OUT_CH), lambda i: (i, 0)),
    out_shape=jax.ShapeDtypeStruct((N_FINE, OUT_CH), jnp.float32),
)


def kernel(residual, down, buffers, W_proj, b_proj, W_skip, b_skip):
    idx = jnp.pad(buffers, (0, SLOTS * NUM_WORKERS * CHUNK - N_FINE))
    idx = idx.reshape(SLOTS, NUM_WORKERS, CHUNK)
    gathered = _sc_gather(idx, down)
    bias = (b_proj + b_skip).reshape(1, OUT_CH)
    return _tc_fused(residual, gathered, W_skip, W_proj, bias)


# speedup vs baseline: 1.9753x; 1.0619x over previous
import functools

import jax
import jax.numpy as jnp
from jax import lax
from jax.experimental import pallas as pl
from jax.experimental.pallas import tpu as pltpu
from jax.experimental.pallas import tpu_sc as plsc

N_FINE = 100000
N_COARSE = 50000
IN_CH = 256
SKIP_CH = 128
OUT_CH = 256

NUM_CORES = 2
NUM_SUBCORES = 16
NUM_WORKERS = NUM_CORES * NUM_SUBCORES

CHUNK = 80
N_CHUNKS = N_FINE // CHUNK
SLOTS = 40


def _sc_gather_body(idx_hbm, down_hbm, out_hbm, idx_all, rows0, rows1,
                    sem_g0, sem_g1, sem_w0, sem_w1):
    wid = lax.axis_index("s") * NUM_CORES + lax.axis_index("c")

    def gather(i, rows, sem):
        return pltpu.make_async_copy(down_hbm.at[idx_all.at[i]], rows, sem)

    def writeback(i, rows, sem):
        c = wid + i * NUM_WORKERS
        return pltpu.make_async_copy(rows, out_hbm.at[pl.ds(c * CHUNK, CHUNK)], sem)

    def valid(i):
        return wid + i * NUM_WORKERS < N_CHUNKS

    pltpu.sync_copy(idx_hbm.at[:, wid], idx_all)
    gather(0, rows0, sem_g0).start()

    def step(t, carry):
        i = 2 * t
        gather(i, rows0, sem_g0).wait()
        writeback(i, rows0, sem_w0).start()

        @pl.when(valid(i + 1))
        def _():
            @pl.when(t > 0)
            def _():
                writeback(i - 1, rows1, sem_w1).wait()

            gather(i + 1, rows1, sem_g1).start()

        @pl.when(valid(i + 1))
        def _():
            gather(i + 1, rows1, sem_g1).wait()
            writeback(i + 1, rows1, sem_w1).start()

        @pl.when(valid(i + 2))
        def _():
            writeback(i, rows0, sem_w0).wait()
            gather(i + 2, rows0, sem_g0).start()

        return carry

    lax.fori_loop(0, SLOTS // 2, step, 0)

    writeback(0, rows0, sem_w0).wait()
    writeback(0, rows1, sem_w1).wait()


_sc_gather = pl.kernel(
    _sc_gather_body,
    out_type=jax.ShapeDtypeStruct((N_FINE, IN_CH), jnp.float32),
    mesh=plsc.VectorSubcoreMesh(core_axis_name="c", subcore_axis_name="s"),
    scratch_types=[
        pltpu.VMEM((SLOTS, CHUNK), jnp.int32),
        pltpu.VMEM((CHUNK, IN_CH), jnp.float32),
        pltpu.VMEM((CHUNK, IN_CH), jnp.float32),
        pltpu.SemaphoreType.DMA,
        pltpu.SemaphoreType.DMA,
        pltpu.SemaphoreType.DMA,
        pltpu.SemaphoreType.DMA,
    ],
)


def _tc_fused_body(res_ref, gat_ref, wskip_ref, wproj_ref, bias_ref, out_ref):
    out_ref[...] = (
        jnp.dot(res_ref[...], wskip_ref[...], preferred_element_type=jnp.float32)
        + jnp.dot(gat_ref[...], wproj_ref[...], preferred_element_type=jnp.float32)
        + bias_ref[...]
    )


ROWS_BLK = 4000
GRID = N_FINE // ROWS_BLK

_tc_fused = pl.pallas_call(
    _tc_fused_body,
    grid=(GRID,),
    in_specs=[
        pl.BlockSpec((ROWS_BLK, SKIP_CH), lambda i: (i, 0)),
        pl.BlockSpec((ROWS_BLK, IN_CH), lambda i: (i, 0)),
        pl.BlockSpec((SKIP_CH, OUT_CH), lambda i: (0, 0)),
        pl.BlockSpec((IN_CH, OUT_CH), lambda i: (0, 0)),
        pl.BlockSpec((1, OUT_CH), lambda i: (0, 0)),
    ],
    out_specs=pl.BlockSpec((ROWS_BLK, OUT_CH), lambda i: (i, 0)),
    out_shape=jax.ShapeDtypeStruct((N_FINE, OUT_CH), jnp.float32),
)


def kernel(residual, down, buffers, W_proj, b_proj, W_skip, b_skip):
    idx = jnp.pad(buffers, (0, SLOTS * NUM_WORKERS * CHUNK - N_FINE))
    idx = idx.reshape(SLOTS, NUM_WORKERS, CHUNK)
    gathered = _sc_gather(idx, down)
    bias = (b_proj + b_skip).reshape(1, OUT_CH)
    return _tc_fused(residual, gathered, W_skip, W_proj, bias)


# speedup vs baseline: 1.9873x; 1.0061x over previous
import functools

import jax
import jax.numpy as jnp
from jax import lax
from jax.experimental import pallas as pl
from jax.experimental.pallas import tpu as pltpu
from jax.experimental.pallas import tpu_sc as plsc

N_FINE = 100000
N_COARSE = 50000
IN_CH = 256
SKIP_CH = 128
OUT_CH = 256

NUM_CORES = 2
NUM_SUBCORES = 16
NUM_WORKERS = NUM_CORES * NUM_SUBCORES

CHUNK = 80
N_CHUNKS = N_FINE // CHUNK
SLOTS = 40


def _sc_gather_body(idx_hbm, down_hbm, out_hbm, idx_all, rows0, rows1,
                    sem_g0, sem_g1, sem_w0, sem_w1):
    wid = lax.axis_index("s") * NUM_CORES + lax.axis_index("c")

    def gather(i, rows, sem):
        return pltpu.make_async_copy(down_hbm.at[idx_all.at[i]], rows, sem)

    def writeback(i, rows, sem):
        c = wid + i * NUM_WORKERS
        return pltpu.make_async_copy(rows, out_hbm.at[pl.ds(c * CHUNK, CHUNK)], sem)

    def valid(i):
        return wid + i * NUM_WORKERS < N_CHUNKS

    pltpu.sync_copy(idx_hbm.at[:, wid], idx_all)
    gather(0, rows0, sem_g0).start()

    def step(t, carry):
        i = 2 * t
        gather(i, rows0, sem_g0).wait()
        writeback(i, rows0, sem_w0).start()

        @pl.when(valid(i + 1))
        def _():
            @pl.when(t > 0)
            def _():
                writeback(i - 1, rows1, sem_w1).wait()

            gather(i + 1, rows1, sem_g1).start()

        @pl.when(valid(i + 1))
        def _():
            gather(i + 1, rows1, sem_g1).wait()
            writeback(i + 1, rows1, sem_w1).start()

        @pl.when(valid(i + 2))
        def _():
            writeback(i, rows0, sem_w0).wait()
            gather(i + 2, rows0, sem_g0).start()

        return carry

    lax.fori_loop(0, SLOTS // 2, step, 0)

    writeback(0, rows0, sem_w0).wait()
    writeback(0, rows1, sem_w1).wait()


_sc_gather = pl.kernel(
    _sc_gather_body,
    out_type=jax.ShapeDtypeStruct((N_FINE, IN_CH), jnp.float32),
    mesh=plsc.VectorSubcoreMesh(core_axis_name="c", subcore_axis_name="s"),
    scratch_types=[
        pltpu.VMEM((SLOTS, CHUNK), jnp.int32),
        pltpu.VMEM((CHUNK, IN_CH), jnp.float32),
        pltpu.VMEM((CHUNK, IN_CH), jnp.float32),
        pltpu.SemaphoreType.DMA,
        pltpu.SemaphoreType.DMA,
        pltpu.SemaphoreType.DMA,
        pltpu.SemaphoreType.DMA,
    ],
)


def _tc_fused_body(res_ref, gat_ref, wskip_ref, wproj_ref, bias_ref, out_ref):
    out_ref[...] = (
        jnp.dot(res_ref[...], wskip_ref[...], preferred_element_type=jnp.float32)
        + jnp.dot(gat_ref[...], wproj_ref[...], preferred_element_type=jnp.float32)
        + bias_ref[...]
    )


ROWS_BLK = 5000
GRID = N_FINE // ROWS_BLK

_tc_fused = pl.pallas_call(
    _tc_fused_body,
    grid=(GRID,),
    in_specs=[
        pl.BlockSpec((ROWS_BLK, SKIP_CH), lambda i: (i, 0)),
        pl.BlockSpec((ROWS_BLK, IN_CH), lambda i: (i, 0)),
        pl.BlockSpec((SKIP_CH, OUT_CH), lambda i: (0, 0)),
        pl.BlockSpec((IN_CH, OUT_CH), lambda i: (0, 0)),
        pl.BlockSpec((1, OUT_CH), lambda i: (0, 0)),
    ],
    out_specs=pl.BlockSpec((ROWS_BLK, OUT_CH), lambda i: (i, 0)),
    out_shape=jax.ShapeDtypeStruct((N_FINE, OUT_CH), jnp.float32),
)


def kernel(residual, down, buffers, W_proj, b_proj, W_skip, b_skip):
    idx = jnp.pad(buffers, (0, SLOTS * NUM_WORKERS * CHUNK - N_FINE))
    idx = idx.reshape(SLOTS, NUM_WORKERS, CHUNK)
    gathered = _sc_gather(idx, down)
    bias = (b_proj + b_skip).reshape(1, OUT_CH)
    return _tc_fused(residual, gathered, W_skip, W_proj, bias)


# speedup vs baseline: 1.9949x; 1.0039x over previous
import functools

import jax
import jax.numpy as jnp
from jax import lax
from jax.experimental import pallas as pl
from jax.experimental.pallas import tpu as pltpu
from jax.experimental.pallas import tpu_sc as plsc

N_FINE = 100000
N_COARSE = 50000
IN_CH = 256
SKIP_CH = 128
OUT_CH = 256

NUM_CORES = 2
NUM_SUBCORES = 16
NUM_WORKERS = NUM_CORES * NUM_SUBCORES

CHUNK = 80
N_CHUNKS = N_FINE // CHUNK
SLOTS = 40


def _sc_gather_body(idx_hbm, down_hbm, out_hbm, idx_all, rows0, rows1,
                    sem_g0, sem_g1, sem_w0, sem_w1):
    wid = lax.axis_index("s") * NUM_CORES + lax.axis_index("c")

    def gather(i, rows, sem):
        return pltpu.make_async_copy(down_hbm.at[idx_all.at[i]], rows, sem)

    def writeback(i, rows, sem):
        c = wid + i * NUM_WORKERS
        return pltpu.make_async_copy(rows, out_hbm.at[pl.ds(c * CHUNK, CHUNK)], sem)

    def valid(i):
        return wid + i * NUM_WORKERS < N_CHUNKS

    pltpu.sync_copy(idx_hbm.at[:, wid], idx_all)
    gather(0, rows0, sem_g0).start()

    def step(t, carry):
        i = 2 * t
        gather(i, rows0, sem_g0).wait()
        writeback(i, rows0, sem_w0).start()

        @pl.when(valid(i + 1))
        def _():
            @pl.when(t > 0)
            def _():
                writeback(i - 1, rows1, sem_w1).wait()

            gather(i + 1, rows1, sem_g1).start()

        @pl.when(valid(i + 1))
        def _():
            gather(i + 1, rows1, sem_g1).wait()
            writeback(i + 1, rows1, sem_w1).start()

        @pl.when(valid(i + 2))
        def _():
            writeback(i, rows0, sem_w0).wait()
            gather(i + 2, rows0, sem_g0).start()

        return carry

    lax.fori_loop(0, SLOTS // 2, step, 0)

    writeback(0, rows0, sem_w0).wait()
    writeback(0, rows1, sem_w1).wait()


_sc_gather = pl.kernel(
    _sc_gather_body,
    out_type=jax.ShapeDtypeStruct((N_FINE, IN_CH), jnp.float32),
    mesh=plsc.VectorSubcoreMesh(core_axis_name="c", subcore_axis_name="s"),
    scratch_types=[
        pltpu.VMEM((SLOTS, CHUNK), jnp.int32),
        pltpu.VMEM((CHUNK, IN_CH), jnp.float32),
        pltpu.VMEM((CHUNK, IN_CH), jnp.float32),
        pltpu.SemaphoreType.DMA,
        pltpu.SemaphoreType.DMA,
        pltpu.SemaphoreType.DMA,
        pltpu.SemaphoreType.DMA,
    ],
)


def _tc_fused_body(res_ref, gat_ref, wskip_ref, wproj_ref, bias_ref, out_ref):
    out_ref[...] = (
        jnp.dot(res_ref[...], wskip_ref[...], preferred_element_type=jnp.float32)
        + jnp.dot(gat_ref[...], wproj_ref[...], preferred_element_type=jnp.float32)
        + bias_ref[...]
    )


ROWS_BLK = 10000
GRID = N_FINE // ROWS_BLK

_tc_fused = pl.pallas_call(
    _tc_fused_body,
    grid=(GRID,),
    in_specs=[
        pl.BlockSpec((ROWS_BLK, SKIP_CH), lambda i: (i, 0)),
        pl.BlockSpec((ROWS_BLK, IN_CH), lambda i: (i, 0)),
        pl.BlockSpec((SKIP_CH, OUT_CH), lambda i: (0, 0)),
        pl.BlockSpec((IN_CH, OUT_CH), lambda i: (0, 0)),
        pl.BlockSpec((1, OUT_CH), lambda i: (0, 0)),
    ],
    out_specs=pl.BlockSpec((ROWS_BLK, OUT_CH), lambda i: (i, 0)),
    out_shape=jax.ShapeDtypeStruct((N_FINE, OUT_CH), jnp.float32),
)


def kernel(residual, down, buffers, W_proj, b_proj, W_skip, b_skip):
    idx = jnp.pad(buffers, (0, SLOTS * NUM_WORKERS * CHUNK - N_FINE))
    idx = idx.reshape(SLOTS, NUM_WORKERS, CHUNK)
    gathered = _sc_gather(idx, down)
    bias = (b_proj + b_skip).reshape(1, OUT_CH)
    return _tc_fused(residual, gathered, W_skip, W_proj, bias)


# speedup vs baseline: 2.0174x; 1.0113x over previous
import functools

import jax
import jax.numpy as jnp
from jax import lax
from jax.experimental import pallas as pl
from jax.experimental.pallas import tpu as pltpu
from jax.experimental.pallas import tpu_sc as plsc

N_FINE = 100000
N_COARSE = 50000
IN_CH = 256
SKIP_CH = 128
OUT_CH = 256

NUM_CORES = 2
NUM_SUBCORES = 16
NUM_WORKERS = NUM_CORES * NUM_SUBCORES

CHUNK = 80
HALF = N_FINE // 2
N_CHUNKS_H = HALF // CHUNK
SLOTS_H = 20


def _sc_gather_body(idx_hbm, down_hbm, out_hbm, idx_all, rows0, rows1,
                    sem_g0, sem_g1, sem_w0, sem_w1):
    wid = lax.axis_index("s") * NUM_CORES + lax.axis_index("c")

    def gather(i, rows, sem):
        return pltpu.make_async_copy(down_hbm.at[idx_all.at[i]], rows, sem)

    def writeback(i, rows, sem):
        c = wid + i * NUM_WORKERS
        return pltpu.make_async_copy(rows, out_hbm.at[pl.ds(c * CHUNK, CHUNK)], sem)

    def valid(i):
        return wid + i * NUM_WORKERS < N_CHUNKS_H

    pltpu.sync_copy(idx_hbm.at[:, wid], idx_all)
    gather(0, rows0, sem_g0).start()

    def step(t, carry):
        i = 2 * t
        gather(i, rows0, sem_g0).wait()
        writeback(i, rows0, sem_w0).start()

        @pl.when(valid(i + 1))
        def _():
            @pl.when(t > 0)
            def _():
                writeback(i - 1, rows1, sem_w1).wait()

            gather(i + 1, rows1, sem_g1).start()

        @pl.when(valid(i + 1))
        def _():
            gather(i + 1, rows1, sem_g1).wait()
            writeback(i + 1, rows1, sem_w1).start()

        @pl.when(valid(i + 2))
        def _():
            writeback(i, rows0, sem_w0).wait()
            gather(i + 2, rows0, sem_g0).start()

        return carry

    lax.fori_loop(0, SLOTS_H // 2, step, 0)

    writeback(0, rows0, sem_w0).wait()
    writeback(0, rows1, sem_w1).wait()


_sc_gather_half = pl.kernel(
    _sc_gather_body,
    out_type=jax.ShapeDtypeStruct((HALF, IN_CH), jnp.float32),
    mesh=plsc.VectorSubcoreMesh(core_axis_name="c", subcore_axis_name="s"),
    scratch_types=[
        pltpu.VMEM((SLOTS_H, CHUNK), jnp.int32),
        pltpu.VMEM((CHUNK, IN_CH), jnp.float32),
        pltpu.VMEM((CHUNK, IN_CH), jnp.float32),
        pltpu.SemaphoreType.DMA,
        pltpu.SemaphoreType.DMA,
        pltpu.SemaphoreType.DMA,
        pltpu.SemaphoreType.DMA,
    ],
)


def _tc_fused_body(res_ref, gat_ref, wskip_ref, wproj_ref, bias_ref, out_ref):
    out_ref[...] = (
        jnp.dot(res_ref[...], wskip_ref[...], preferred_element_type=jnp.float32)
        + jnp.dot(gat_ref[...], wproj_ref[...], preferred_element_type=jnp.float32)
        + bias_ref[...]
    )


def _tc_fused_body2(res_ref, gat_ref, wskip_ref, wproj_ref, bias_ref, part_ref,
                    out_ref):
    del part_ref
    _tc_fused_body(res_ref, gat_ref, wskip_ref, wproj_ref, bias_ref, out_ref)


ROWS_BLK = 5000
GRID_H = HALF // ROWS_BLK

_W_SPECS = [
    pl.BlockSpec((SKIP_CH, OUT_CH), lambda i: (0, 0)),
    pl.BlockSpec((IN_CH, OUT_CH), lambda i: (0, 0)),
    pl.BlockSpec((1, OUT_CH), lambda i: (0, 0)),
]

_tc_first = pl.pallas_call(
    _tc_fused_body,
    grid=(GRID_H,),
    in_specs=[
        pl.BlockSpec((ROWS_BLK, SKIP_CH), lambda i: (i, 0)),
        pl.BlockSpec((ROWS_BLK, IN_CH), lambda i: (i, 0)),
        *_W_SPECS,
    ],
    out_specs=pl.BlockSpec((ROWS_BLK, OUT_CH), lambda i: (i, 0)),
    out_shape=jax.ShapeDtypeStruct((N_FINE, OUT_CH), jnp.float32),
)

_tc_second = pl.pallas_call(
    _tc_fused_body2,
    grid=(GRID_H,),
    in_specs=[
        pl.BlockSpec((ROWS_BLK, SKIP_CH), lambda i: (i + GRID_H, 0)),
        pl.BlockSpec((ROWS_BLK, IN_CH), lambda i: (i, 0)),
        *_W_SPECS,
        pl.BlockSpec(memory_space=pl.ANY),
    ],
    out_specs=pl.BlockSpec((ROWS_BLK, OUT_CH), lambda i: (i + GRID_H, 0)),
    out_shape=jax.ShapeDtypeStruct((N_FINE, OUT_CH), jnp.float32),
    input_output_aliases={5: 0},
)


def _half_idx(buffers_half):
    pad = SLOTS_H * NUM_WORKERS * CHUNK - HALF
    return jnp.pad(buffers_half, (0, pad)).reshape(SLOTS_H, NUM_WORKERS, CHUNK)


def kernel(residual, down, buffers, W_proj, b_proj, W_skip, b_skip):
    bias = (b_proj + b_skip).reshape(1, OUT_CH)
    g0 = _sc_gather_half(_half_idx(buffers[:HALF]), down)
    g1 = _sc_gather_half(_half_idx(buffers[HALF:]), down)
    part = _tc_first(residual, g0, W_skip, W_proj, bias)
    return _tc_second(residual, g1, W_skip, W_proj, bias, part)
